# baseline (speedup 1.0000x reference)
import jax
import jax.numpy as jnp
from jax.experimental import pallas as pl
from jax.experimental.pallas import tpu as pltpu

_N = 20000
_C = 91
_K = 100
_KPAD = 128
_CH = 2000


def _post_process_kernel(logits_ref, boxes_ref, scale_ref,
                         scores_ref, labels_ref, boxes_out_ref):
    big_i32 = jnp.int32(2**31 - 1)
    neg_inf = jnp.float32(-jnp.inf)

    pieces = []
    for c in range(_N // _CH):
        v = logits_ref[0, c * _CH:(c + 1) * _CH, :]
        pieces.append(jnp.max(v, axis=1, keepdims=True).reshape(1, _CH))
    rowmax = jnp.concatenate(pieces, axis=1)

    row_iota = jax.lax.broadcasted_iota(jnp.int32, (1, _N), 1)
    crow_iota = jax.lax.broadcasted_iota(jnp.int32, (_K, 1), 0)

    def sel_row(i, carry):
        bm, cand, rowid = carry
        m = jnp.max(bm)
        n = jnp.min(jnp.where(bm == m, row_iota, big_i32))
        bm = jnp.where(row_iota == n, neg_inf, bm)
        xrow = logits_ref[0, pl.ds(n, 1), :]
        hit = crow_iota == i
        cand = jnp.where(hit, xrow, cand)
        rowid = jnp.where(hit, n, rowid)
        return bm, cand, rowid

    cand0 = jnp.full((_K, _C), neg_inf, dtype=jnp.float32)
    rowid0 = jnp.zeros((_K, 1), dtype=jnp.int32)
    _, cand, rowid = jax.lax.fori_loop(
        0, _K, sel_row, (rowmax, cand0, rowid0))

    lane_iota_c = jax.lax.broadcasted_iota(jnp.int32, (_K, _C), 1)
    idxmap = rowid * _C + lane_iota_c

    out_lane = jax.lax.broadcasted_iota(jnp.int32, (1, _KPAD), 1)
    brow_iota = jax.lax.broadcasted_iota(jnp.int32, (_KPAD, 1), 0)
    bcol_iota = jax.lax.broadcasted_iota(jnp.int32, (1, 4), 1)
    lane32 = jax.lax.broadcasted_iota(jnp.int32, (1, 32), 1)

    def sel_elem(i, carry):
        cd, sc, lb, bo = carry
        m = jnp.max(cd)
        gf = jnp.min(jnp.where(cd == m, idxmap, big_i32))
        cd = jnp.where(idxmap == gf, neg_inf, cd)
        n2 = gf // _C
        lab = gf - n2 * _C
        hit = out_lane == i
        sc = jnp.where(hit, m, sc)
        lb = jnp.where(hit, lab, lb)
        r = n2 // 8
        q = n2 - 8 * r
        row32 = boxes_ref[0, pl.ds(r, 1), :]
        hit_row = brow_iota == i
        for j in range(4):
            cj = jnp.sum(jnp.where(lane32 == 4 * q + j, row32, 0.0))
            bo = jnp.where(hit_row & (bcol_iota == j), cj, bo)
        return cd, sc, lb, bo

    sc0 = jnp.zeros((1, _KPAD), dtype=jnp.float32)
    lb0 = jnp.zeros((1, _KPAD), dtype=jnp.int32)
    bo0 = jnp.zeros((_KPAD, 4), dtype=jnp.float32)
    _, sc, lb, bo = jax.lax.fori_loop(
        0, _K, sel_elem, (cand, sc0, lb0, bo0))

    scores_ref[0, 0, :] = jax.nn.sigmoid(sc[0])
    labels_ref[0, 0, :] = lb[0]
    cx = bo[:, 0:1]
    cy = bo[:, 1:2]
    w = bo[:, 2:3]
    h = bo[:, 3:4]
    xyxy = jnp.concatenate(
        [cx - 0.5 * w, cy - 0.5 * h, cx + 0.5 * w, cy + 0.5 * h], axis=1)
    boxes_out_ref[0] = xyxy * scale_ref[0, 0]


@jax.jit
def kernel(pred_logits, pred_boxes, target_sizes):
    b = pred_logits.shape[0]
    img_h = target_sizes[:, 0].astype(jnp.float32)
    img_w = target_sizes[:, 1].astype(jnp.float32)
    scale_fct = jnp.stack([img_w, img_h, img_w, img_h], axis=1)
    scale_fct = scale_fct.reshape(b, 1, 4)
    boxes_packed = pred_boxes.reshape(b, _N // 8, 32)

    scores_p, labels_p, boxes_p = pl.pallas_call(
        _post_process_kernel,
        grid=(b,),
        in_specs=[
            pl.BlockSpec((1, _N, _C), lambda i: (i, 0, 0)),
            pl.BlockSpec((1, _N // 8, 32), lambda i: (i, 0, 0)),
            pl.BlockSpec((1, 1, 4), lambda i: (i, 0, 0)),
        ],
        out_specs=[
            pl.BlockSpec((1, 1, _KPAD), lambda i: (i, 0, 0)),
            pl.BlockSpec((1, 1, _KPAD), lambda i: (i, 0, 0)),
            pl.BlockSpec((1, _KPAD, 4), lambda i: (i, 0, 0)),
        ],
        out_shape=[
            jax.ShapeDtypeStruct((b, 1, _KPAD), jnp.float32),
            jax.ShapeDtypeStruct((b, 1, _KPAD), jnp.int32),
            jax.ShapeDtypeStruct((b, _KPAD, 4), jnp.float32),
        ],
        compiler_params=pltpu.CompilerParams(
            dimension_semantics=("parallel",)),
    )(pred_logits, boxes_packed, scale_fct)

    return scores_p[:, 0, :_K], labels_p[:, 0, :_K], boxes_p[:, :_K, :]


# speedup vs baseline: 1.0633x; 1.0633x over previous
import jax
import jax.numpy as jnp
from jax.experimental import pallas as pl
from jax.experimental.pallas import tpu as pltpu

_N = 20000
_C = 91
_K = 100
_KPAD = 128
_CH = 2000


def _post_process_kernel(logits_ref, boxes_ref, scale_ref,
                         scores_ref, labels_ref, boxes_out_ref):
    big_i32 = jnp.int32(2**31 - 1)
    neg_inf = jnp.float32(-jnp.inf)

    pieces = []
    for c in range(8):
        v = logits_ref[0, c * (_N // 8):(c + 1) * (_N // 8), :]
        pieces.append(
            jnp.max(v, axis=1, keepdims=True).reshape(1, _N // 8))
    rowmax = jnp.concatenate(pieces, axis=0)

    grp_idx = (jax.lax.broadcasted_iota(jnp.int32, (8, _N // 8), 0)
               * (_N // 8)
               + jax.lax.broadcasted_iota(jnp.int32, (8, _N // 8), 1))
    lane_k = jax.lax.broadcasted_iota(jnp.int32, (1, _KPAD), 1)
    crow_iota = jax.lax.broadcasted_iota(jnp.int32, (_K, 1), 0)

    def sel_grp(i, carry):
        bm, vals, ids = carry
        m = jnp.max(bm, axis=1, keepdims=True)
        sel = jnp.min(jnp.where(bm == m, grp_idx, big_i32),
                      axis=1, keepdims=True)
        bm = jnp.where(grp_idx == sel, neg_inf, bm)
        hit = lane_k == i
        vals = jnp.where(hit, m, vals)
        ids = jnp.where(hit, sel, ids)
        return bm, vals, ids

    vals0 = jnp.full((8, _KPAD), neg_inf, dtype=jnp.float32)
    ids0 = jnp.full((8, _KPAD), -1, dtype=jnp.int32)
    _, vals, ids = jax.lax.fori_loop(
        0, _K, sel_grp, (rowmax, vals0, ids0))

    def sel_row(i, carry):
        vs, cand, rowid = carry
        m = jnp.max(vs)
        n = jnp.min(jnp.where(vs == m, ids, big_i32))
        vs = jnp.where(ids == n, neg_inf, vs)
        xrow = logits_ref[0, pl.ds(n, 1), :]
        hit = crow_iota == i
        cand = jnp.where(hit, xrow, cand)
        rowid = jnp.where(hit, n, rowid)
        return vs, cand, rowid

    cand0 = jnp.full((_K, _C), neg_inf, dtype=jnp.float32)
    rowid0 = jnp.zeros((_K, 1), dtype=jnp.int32)
    _, cand, rowid = jax.lax.fori_loop(
        0, _K, sel_row, (vals, cand0, rowid0))

    lane_iota_c = jax.lax.broadcasted_iota(jnp.int32, (_K, _C), 1)
    idxmap = rowid * _C + lane_iota_c

    out_lane = jax.lax.broadcasted_iota(jnp.int32, (1, _KPAD), 1)
    brow_iota = jax.lax.broadcasted_iota(jnp.int32, (_KPAD, 1), 0)
    bcol_iota = jax.lax.broadcasted_iota(jnp.int32, (1, 4), 1)
    lane32 = jax.lax.broadcasted_iota(jnp.int32, (1, 32), 1)

    def sel_elem(i, carry):
        cd, sc, lb, bo = carry
        m = jnp.max(cd)
        gf = jnp.min(jnp.where(cd == m, idxmap, big_i32))
        cd = jnp.where(idxmap == gf, neg_inf, cd)
        n2 = gf // _C
        lab = gf - n2 * _C
        hit = out_lane == i
        sc = jnp.where(hit, m, sc)
        lb = jnp.where(hit, lab, lb)
        r = n2 // 8
        q = n2 - 8 * r
        row32 = boxes_ref[0, pl.ds(r, 1), :]
        hit_row = brow_iota == i
        for j in range(4):
            cj = jnp.sum(jnp.where(lane32 == 4 * q + j, row32, 0.0))
            bo = jnp.where(hit_row & (bcol_iota == j), cj, bo)
        return cd, sc, lb, bo

    sc0 = jnp.zeros((1, _KPAD), dtype=jnp.float32)
    lb0 = jnp.zeros((1, _KPAD), dtype=jnp.int32)
    bo0 = jnp.zeros((_KPAD, 4), dtype=jnp.float32)
    _, sc, lb, bo = jax.lax.fori_loop(
        0, _K, sel_elem, (cand, sc0, lb0, bo0))

    scores_ref[0, 0, :] = jax.nn.sigmoid(sc[0])
    labels_ref[0, 0, :] = lb[0]
    cx = bo[:, 0:1]
    cy = bo[:, 1:2]
    w = bo[:, 2:3]
    h = bo[:, 3:4]
    xyxy = jnp.concatenate(
        [cx - 0.5 * w, cy - 0.5 * h, cx + 0.5 * w, cy + 0.5 * h], axis=1)
    boxes_out_ref[0] = xyxy * scale_ref[0, 0]


@jax.jit
def kernel(pred_logits, pred_boxes, target_sizes):
    b = pred_logits.shape[0]
    img_h = target_sizes[:, 0].astype(jnp.float32)
    img_w = target_sizes[:, 1].astype(jnp.float32)
    scale_fct = jnp.stack([img_w, img_h, img_w, img_h], axis=1)
    scale_fct = scale_fct.reshape(b, 1, 4)
    boxes_packed = pred_boxes.reshape(b, _N // 8, 32)

    scores_p, labels_p, boxes_p = pl.pallas_call(
        _post_process_kernel,
        grid=(b,),
        in_specs=[
            pl.BlockSpec((1, _N, _C), lambda i: (i, 0, 0)),
            pl.BlockSpec((1, _N // 8, 32), lambda i: (i, 0, 0)),
            pl.BlockSpec((1, 1, 4), lambda i: (i, 0, 0)),
        ],
        out_specs=[
            pl.BlockSpec((1, 1, _KPAD), lambda i: (i, 0, 0)),
            pl.BlockSpec((1, 1, _KPAD), lambda i: (i, 0, 0)),
            pl.BlockSpec((1, _KPAD, 4), lambda i: (i, 0, 0)),
        ],
        out_shape=[
            jax.ShapeDtypeStruct((b, 1, _KPAD), jnp.float32),
            jax.ShapeDtypeStruct((b, 1, _KPAD), jnp.int32),
            jax.ShapeDtypeStruct((b, _KPAD, 4), jnp.float32),
        ],
        compiler_params=pltpu.CompilerParams(
            dimension_semantics=("parallel",)),
    )(pred_logits, boxes_packed, scale_fct)

    return scores_p[:, 0, :_K], labels_p[:, 0, :_K], boxes_p[:, :_K, :]


# speedup vs baseline: 1.1607x; 1.0916x over previous
import jax
import jax.numpy as jnp
from jax.experimental import pallas as pl
from jax.experimental.pallas import tpu as pltpu

_N = 20000
_C = 91
_K = 100
_KPAD = 128
_CH = 2000


def _post_process_kernel(logits_ref, boxes_ref, scale_ref,
                         scores_ref, labels_ref, boxes_out_ref):
    big_i32 = jnp.int32(2**31 - 1)
    neg_inf = jnp.float32(-jnp.inf)

    pieces = []
    for c in range(8):
        v = logits_ref[0, c * (_N // 8):(c + 1) * (_N // 8), :]
        pieces.append(
            jnp.max(v, axis=1, keepdims=True).reshape(1, _N // 8))
    rowmax = jnp.concatenate(pieces, axis=0)

    grp_idx = (jax.lax.broadcasted_iota(jnp.int32, (8, _N // 8), 0)
               * (_N // 8)
               + jax.lax.broadcasted_iota(jnp.int32, (8, _N // 8), 1))
    lane_k = jax.lax.broadcasted_iota(jnp.int32, (1, _KPAD), 1)
    crow_iota = jax.lax.broadcasted_iota(jnp.int32, (_K, 1), 0)

    def sel_grp(i, carry):
        bm, vals, ids = carry
        m = jnp.max(bm, axis=1, keepdims=True)
        sel = jnp.min(jnp.where(bm == m, grp_idx, big_i32),
                      axis=1, keepdims=True)
        bm = jnp.where(grp_idx == sel, neg_inf, bm)
        hit = lane_k == i
        vals = jnp.where(hit, m, vals)
        ids = jnp.where(hit, sel, ids)
        return bm, vals, ids

    vals0 = jnp.full((8, _KPAD), neg_inf, dtype=jnp.float32)
    ids0 = jnp.full((8, _KPAD), -1, dtype=jnp.int32)
    _, vals, ids = jax.lax.fori_loop(
        0, _K, sel_grp, (rowmax, vals0, ids0), unroll=4)

    def sel_row(i, carry):
        vs, cand, rowid = carry
        m = jnp.max(vs)
        n = jnp.min(jnp.where(vs == m, ids, big_i32))
        vs = jnp.where(ids == n, neg_inf, vs)
        xrow = logits_ref[0, pl.ds(n, 1), :]
        hit = crow_iota == i
        cand = jnp.where(hit, xrow, cand)
        rowid = jnp.where(hit, n, rowid)
        return vs, cand, rowid

    cand0 = jnp.full((_K, _C), neg_inf, dtype=jnp.float32)
    rowid0 = jnp.zeros((_K, 1), dtype=jnp.int32)
    _, cand, rowid = jax.lax.fori_loop(
        0, _K, sel_row, (vals, cand0, rowid0), unroll=4)

    lane_iota_c = jax.lax.broadcasted_iota(jnp.int32, (_K, _C), 1)
    idxmap = rowid * _C + lane_iota_c

    out_lane = jax.lax.broadcasted_iota(jnp.int32, (1, _KPAD), 1)
    brow_iota = jax.lax.broadcasted_iota(jnp.int32, (_KPAD, 1), 0)
    bcol_iota = jax.lax.broadcasted_iota(jnp.int32, (1, 4), 1)
    lane32 = jax.lax.broadcasted_iota(jnp.int32, (1, 32), 1)

    def sel_elem(i, carry):
        cd, sc, lb, bo = carry
        m = jnp.max(cd)
        gf = jnp.min(jnp.where(cd == m, idxmap, big_i32))
        cd = jnp.where(idxmap == gf, neg_inf, cd)
        n2 = gf // _C
        lab = gf - n2 * _C
        hit = out_lane == i
        sc = jnp.where(hit, m, sc)
        lb = jnp.where(hit, lab, lb)
        r = n2 // 8
        q = n2 - 8 * r
        row32 = boxes_ref[0, pl.ds(r, 1), :]
        hit_row = brow_iota == i
        for j in range(4):
            cj = jnp.sum(jnp.where(lane32 == 4 * q + j, row32, 0.0))
            bo = jnp.where(hit_row & (bcol_iota == j), cj, bo)
        return cd, sc, lb, bo

    sc0 = jnp.zeros((1, _KPAD), dtype=jnp.float32)
    lb0 = jnp.zeros((1, _KPAD), dtype=jnp.int32)
    bo0 = jnp.zeros((_KPAD, 4), dtype=jnp.float32)
    _, sc, lb, bo = jax.lax.fori_loop(
        0, _K, sel_elem, (cand, sc0, lb0, bo0), unroll=4)

    scores_ref[0, 0, :] = jax.nn.sigmoid(sc[0])
    labels_ref[0, 0, :] = lb[0]
    cx = bo[:, 0:1]
    cy = bo[:, 1:2]
    w = bo[:, 2:3]
    h = bo[:, 3:4]
    xyxy = jnp.concatenate(
        [cx - 0.5 * w, cy - 0.5 * h, cx + 0.5 * w, cy + 0.5 * h], axis=1)
    boxes_out_ref[0] = xyxy * scale_ref[0, 0]


@jax.jit
def kernel(pred_logits, pred_boxes, target_sizes):
    b = pred_logits.shape[0]
    img_h = target_sizes[:, 0].astype(jnp.float32)
    img_w = target_sizes[:, 1].astype(jnp.float32)
    scale_fct = jnp.stack([img_w, img_h, img_w, img_h], axis=1)
    scale_fct = scale_fct.reshape(b, 1, 4)
    boxes_packed = pred_boxes.reshape(b, _N // 8, 32)

    scores_p, labels_p, boxes_p = pl.pallas_call(
        _post_process_kernel,
        grid=(b,),
        in_specs=[
            pl.BlockSpec((1, _N, _C), lambda i: (i, 0, 0)),
            pl.BlockSpec((1, _N // 8, 32), lambda i: (i, 0, 0)),
            pl.BlockSpec((1, 1, 4), lambda i: (i, 0, 0)),
        ],
        out_specs=[
            pl.BlockSpec((1, 1, _KPAD), lambda i: (i, 0, 0)),
            pl.BlockSpec((1, 1, _KPAD), lambda i: (i, 0, 0)),
            pl.BlockSpec((1, _KPAD, 4), lambda i: (i, 0, 0)),
        ],
        out_shape=[
            jax.ShapeDtypeStruct((b, 1, _KPAD), jnp.float32),
            jax.ShapeDtypeStruct((b, 1, _KPAD), jnp.int32),
            jax.ShapeDtypeStruct((b, _KPAD, 4), jnp.float32),
        ],
        compiler_params=pltpu.CompilerParams(
            dimension_semantics=("parallel",)),
    )(pred_logits, boxes_packed, scale_fct)

    return scores_p[:, 0, :_K], labels_p[:, 0, :_K], boxes_p[:, :_K, :]


# speedup vs baseline: 1.1754x; 1.0127x over previous
import jax
import jax.numpy as jnp
from jax.experimental import pallas as pl
from jax.experimental.pallas import tpu as pltpu

_N = 20000
_C = 91
_K = 100
_KPAD = 128
_CH = 2000


def _post_process_kernel(logits_ref, boxes_ref, scale_ref,
                         scores_ref, labels_ref, boxes_out_ref):
    big_i32 = jnp.int32(2**31 - 1)
    neg_inf = jnp.float32(-jnp.inf)

    pieces = []
    for c in range(8):
        v = logits_ref[0, c * (_N // 8):(c + 1) * (_N // 8), :]
        pieces.append(
            jnp.max(v, axis=1, keepdims=True).reshape(1, _N // 8))
    rowmax = jnp.concatenate(pieces, axis=0)

    grp_idx = (jax.lax.broadcasted_iota(jnp.int32, (8, _N // 8), 0)
               * (_N // 8)
               + jax.lax.broadcasted_iota(jnp.int32, (8, _N // 8), 1))
    lane_k = jax.lax.broadcasted_iota(jnp.int32, (1, _KPAD), 1)
    crow_iota = jax.lax.broadcasted_iota(jnp.int32, (_K, 1), 0)

    def sel_grp(i, carry):
        bm, vals, ids = carry
        m = jnp.max(bm, axis=1, keepdims=True)
        sel = jnp.min(jnp.where(bm == m, grp_idx, big_i32),
                      axis=1, keepdims=True)
        bm = jnp.where(grp_idx == sel, neg_inf, bm)
        hit = lane_k == i
        vals = jnp.where(hit, m, vals)
        ids = jnp.where(hit, sel, ids)
        return bm, vals, ids

    vals0 = jnp.full((8, _KPAD), neg_inf, dtype=jnp.float32)
    ids0 = jnp.full((8, _KPAD), -1, dtype=jnp.int32)
    _, vals, ids = jax.lax.fori_loop(
        0, _K, sel_grp, (rowmax, vals0, ids0), unroll=8)

    def sel_row(i, carry):
        vs, cand, rowid = carry
        m = jnp.max(vs)
        n = jnp.min(jnp.where(vs == m, ids, big_i32))
        vs = jnp.where(ids == n, neg_inf, vs)
        xrow = logits_ref[0, pl.ds(n, 1), :]
        hit = crow_iota == i
        cand = jnp.where(hit, xrow, cand)
        rowid = jnp.where(hit, n, rowid)
        return vs, cand, rowid

    cand0 = jnp.full((_K, _C), neg_inf, dtype=jnp.float32)
    rowid0 = jnp.zeros((_K, 1), dtype=jnp.int32)
    _, cand, rowid = jax.lax.fori_loop(
        0, _K, sel_row, (vals, cand0, rowid0), unroll=8)

    lane_iota_c = jax.lax.broadcasted_iota(jnp.int32, (_K, _C), 1)
    idxmap = rowid * _C + lane_iota_c

    out_lane = jax.lax.broadcasted_iota(jnp.int32, (1, _KPAD), 1)
    brow_iota = jax.lax.broadcasted_iota(jnp.int32, (_KPAD, 1), 0)
    bcol_iota = jax.lax.broadcasted_iota(jnp.int32, (1, 4), 1)
    lane32 = jax.lax.broadcasted_iota(jnp.int32, (1, 32), 1)

    def sel_elem(i, carry):
        cd, sc, lb, bo = carry
        m = jnp.max(cd)
        gf = jnp.min(jnp.where(cd == m, idxmap, big_i32))
        cd = jnp.where(idxmap == gf, neg_inf, cd)
        n2 = gf // _C
        lab = gf - n2 * _C
        hit = out_lane == i
        sc = jnp.where(hit, m, sc)
        lb = jnp.where(hit, lab, lb)
        r = n2 // 8
        q = n2 - 8 * r
        row32 = boxes_ref[0, pl.ds(r, 1), :]
        hit_row = brow_iota == i
        for j in range(4):
            cj = jnp.sum(jnp.where(lane32 == 4 * q + j, row32, 0.0))
            bo = jnp.where(hit_row & (bcol_iota == j), cj, bo)
        return cd, sc, lb, bo

    sc0 = jnp.zeros((1, _KPAD), dtype=jnp.float32)
    lb0 = jnp.zeros((1, _KPAD), dtype=jnp.int32)
    bo0 = jnp.zeros((_KPAD, 4), dtype=jnp.float32)
    _, sc, lb, bo = jax.lax.fori_loop(
        0, _K, sel_elem, (cand, sc0, lb0, bo0), unroll=8)

    scores_ref[0, 0, :] = jax.nn.sigmoid(sc[0])
    labels_ref[0, 0, :] = lb[0]
    cx = bo[:, 0:1]
    cy = bo[:, 1:2]
    w = bo[:, 2:3]
    h = bo[:, 3:4]
    xyxy = jnp.concatenate(
        [cx - 0.5 * w, cy - 0.5 * h, cx + 0.5 * w, cy + 0.5 * h], axis=1)
    boxes_out_ref[0] = xyxy * scale_ref[0, 0]


@jax.jit
def kernel(pred_logits, pred_boxes, target_sizes):
    b = pred_logits.shape[0]
    img_h = target_sizes[:, 0].astype(jnp.float32)
    img_w = target_sizes[:, 1].astype(jnp.float32)
    scale_fct = jnp.stack([img_w, img_h, img_w, img_h], axis=1)
    scale_fct = scale_fct.reshape(b, 1, 4)
    boxes_packed = pred_boxes.reshape(b, _N // 8, 32)

    scores_p, labels_p, boxes_p = pl.pallas_call(
        _post_process_kernel,
        grid=(b,),
        in_specs=[
            pl.BlockSpec((1, _N, _C), lambda i: (i, 0, 0)),
            pl.BlockSpec((1, _N // 8, 32), lambda i: (i, 0, 0)),
            pl.BlockSpec((1, 1, 4), lambda i: (i, 0, 0)),
        ],
        out_specs=[
            pl.BlockSpec((1, 1, _KPAD), lambda i: (i, 0, 0)),
            pl.BlockSpec((1, 1, _KPAD), lambda i: (i, 0, 0)),
            pl.BlockSpec((1, _KPAD, 4), lambda i: (i, 0, 0)),
        ],
        out_shape=[
            jax.ShapeDtypeStruct((b, 1, _KPAD), jnp.float32),
            jax.ShapeDtypeStruct((b, 1, _KPAD), jnp.int32),
            jax.ShapeDtypeStruct((b, _KPAD, 4), jnp.float32),
        ],
        compiler_params=pltpu.CompilerParams(
            dimension_semantics=("parallel",)),
    )(pred_logits, boxes_packed, scale_fct)

    return scores_p[:, 0, :_K], labels_p[:, 0, :_K], boxes_p[:, :_K, :]


# speedup vs baseline: 1.1756x; 1.0002x over previous
import jax
import jax.numpy as jnp
from jax.experimental import pallas as pl
from jax.experimental.pallas import tpu as pltpu

_N = 20000
_C = 91
_K = 100
_KPAD = 128


def _post_process_kernel(logits_ref, boxes_ref, scale_ref,
                         scores_ref, labels_ref, boxes_out_ref):
    big_i32 = jnp.int32(2**31 - 1)
    neg_inf = jnp.float32(-jnp.inf)

    pieces = []
    for c in range(8):
        v = logits_ref[0, c * (_N // 8):(c + 1) * (_N // 8), :]
        pieces.append(
            jnp.max(v, axis=1, keepdims=True).reshape(1, _N // 8))
    rowmax = jnp.concatenate(pieces, axis=0)

    grp_idx = (jax.lax.broadcasted_iota(jnp.int32, (8, _N // 8), 0)
               * (_N // 8)
               + jax.lax.broadcasted_iota(jnp.int32, (8, _N // 8), 1))
    lane_k = jax.lax.broadcasted_iota(jnp.int32, (1, _KPAD), 1)
    crow_iota = jax.lax.broadcasted_iota(jnp.int32, (_K, 1), 0)

    def sel_grp(i, carry):
        bm, vals, ids = carry
        m = jnp.max(bm, axis=1, keepdims=True)
        sel = jnp.min(jnp.where(bm == m, grp_idx, big_i32),
                      axis=1, keepdims=True)
        bm = jnp.where(grp_idx == sel, neg_inf, bm)
        hit = lane_k == i
        vals = jnp.where(hit, m, vals)
        ids = jnp.where(hit, sel, ids)
        return bm, vals, ids

    vals0 = jnp.full((8, _KPAD), neg_inf, dtype=jnp.float32)
    ids0 = jnp.full((8, _KPAD), -1, dtype=jnp.int32)
    _, vals, ids = jax.lax.fori_loop(
        0, _K, sel_grp, (rowmax, vals0, ids0), unroll=8)

    def sel_row(i, carry):
        vs, cand, rowid = carry
        m = jnp.max(vs)
        n = jnp.min(jnp.where(vs == m, ids, big_i32))
        vs = jnp.where(ids == n, neg_inf, vs)
        xrow = logits_ref[0, pl.ds(n, 1), :]
        hit = crow_iota == i
        cand = jnp.where(hit, xrow, cand)
        rowid = jnp.where(hit, n, rowid)
        return vs, cand, rowid

    cand0 = jnp.full((_K, _C), neg_inf, dtype=jnp.float32)
    rowid0 = jnp.zeros((_K, 1), dtype=jnp.int32)
    _, cand, rowid = jax.lax.fori_loop(
        0, _K, sel_row, (vals, cand0, rowid0), unroll=8)

    lane_iota_c = jax.lax.broadcasted_iota(jnp.int32, (_K, _C), 1)
    idxmap = rowid * _C + lane_iota_c

    out_lane = jax.lax.broadcasted_iota(jnp.int32, (1, _KPAD), 1)
    brow_iota = jax.lax.broadcasted_iota(jnp.int32, (_KPAD, 1), 0)
    bcol_iota = jax.lax.broadcasted_iota(jnp.int32, (1, 4), 1)
    lane32 = jax.lax.broadcasted_iota(jnp.int32, (1, 32), 1)

    def sel_elem(i, carry):
        cd, sc, lb, bo = carry
        m = jnp.max(cd)
        gf = jnp.min(jnp.where(cd == m, idxmap, big_i32))
        cd = jnp.where(idxmap == gf, neg_inf, cd)
        n2 = gf // _C
        lab = gf - n2 * _C
        hit = out_lane == i
        sc = jnp.where(hit, m, sc)
        lb = jnp.where(hit, lab, lb)
        r = n2 // 8
        q = n2 - 8 * r
        row32 = boxes_ref[0, pl.ds(r, 1), :]
        hit_row = brow_iota == i
        for j in range(4):
            cj = jnp.sum(jnp.where(lane32 == 4 * q + j, row32, 0.0))
            bo = jnp.where(hit_row & (bcol_iota == j), cj, bo)
        return cd, sc, lb, bo

    sc0 = jnp.zeros((1, _KPAD), dtype=jnp.float32)
    lb0 = jnp.zeros((1, _KPAD), dtype=jnp.int32)
    bo0 = jnp.zeros((_KPAD, 4), dtype=jnp.float32)
    _, sc, lb, bo = jax.lax.fori_loop(
        0, _K, sel_elem, (cand, sc0, lb0, bo0), unroll=8)

    scores_ref[0, 0, :] = jax.nn.sigmoid(sc[0])
    labels_ref[0, 0, :] = lb[0]
    cx = bo[:, 0:1]
    cy = bo[:, 1:2]
    w = bo[:, 2:3]
    h = bo[:, 3:4]
    xyxy = jnp.concatenate(
        [cx - 0.5 * w, cy - 0.5 * h, cx + 0.5 * w, cy + 0.5 * h], axis=1)
    boxes_out_ref[0] = xyxy * scale_ref[0, 0]


@jax.jit
def kernel(pred_logits, pred_boxes, target_sizes):
    b = pred_logits.shape[0]
    img_h = target_sizes[:, 0].astype(jnp.float32)
    img_w = target_sizes[:, 1].astype(jnp.float32)
    scale_fct = jnp.stack([img_w, img_h, img_w, img_h], axis=1)
    scale_fct = scale_fct.reshape(b, 1, 4)
    boxes_packed = pred_boxes.reshape(b, _N // 8, 32)

    scores_p, labels_p, boxes_p = pl.pallas_call(
        _post_process_kernel,
        grid=(b,),
        in_specs=[
            pl.BlockSpec((1, _N, _C), lambda i: (i, 0, 0)),
            pl.BlockSpec((1, _N // 8, 32), lambda i: (i, 0, 0)),
            pl.BlockSpec((1, 1, 4), lambda i: (i, 0, 0)),
        ],
        out_specs=[
            pl.BlockSpec((1, 1, _KPAD), lambda i: (i, 0, 0)),
            pl.BlockSpec((1, 1, _KPAD), lambda i: (i, 0, 0)),
            pl.BlockSpec((1, _KPAD, 4), lambda i: (i, 0, 0)),
        ],
        out_shape=[
            jax.ShapeDtypeStruct((b, 1, _KPAD), jnp.float32),
            jax.ShapeDtypeStruct((b, 1, _KPAD), jnp.int32),
            jax.ShapeDtypeStruct((b, _KPAD, 4), jnp.float32),
        ],
        compiler_params=pltpu.CompilerParams(
            dimension_semantics=("parallel",)),
    )(pred_logits, boxes_packed, scale_fct)

    return scores_p[:, 0, :_K], labels_p[:, 0, :_K], boxes_p[:, :_K, :]
